# baseline (speedup 1.0000x reference)
import functools

import jax
import jax.numpy as jnp
from jax import lax
from jax.experimental import pallas as pl
from jax.experimental.pallas import tpu as pltpu
from jax.experimental.pallas import tpu_sc as plsc

N = 10000
NP = 10240
E = 320000
D = 128
NC = 2
NS = 16
NW = NC * NS
CH = 64
EW = E // NW
CF = EW // CH
CT = EW - CF * CH
DR = 160
BT = 2048
NB = NP // BT
NBUF = 4
NIB = 6

_vmesh = plsc.VectorSubcoreMesh(core_axis_name="c", subcore_axis_name="s")


@functools.partial(
    pl.kernel,
    out_type=jax.ShapeDtypeStruct((NC, NP), jnp.float32),
    mesh=_vmesh,
    scratch_types=[
        pltpu.VMEM_SHARED((NP,), jnp.float32),
        pltpu.VMEM((DR, CH), jnp.int32),
        pltpu.VMEM((CH,), jnp.float32),
        pltpu.VMEM((NP // NS,), jnp.float32),
        pltpu.SemaphoreType.DMA,
    ],
)
def _deg_kernel(e2_hbm, out_hbm, acc, didx, onesv, zerov, sem):
    c = lax.axis_index("c")
    s = lax.axis_index("s")
    w = s * NC + c

    @pl.loop(0, CH, step=16)
    def _(i):
        onesv[pl.ds(i, 16)] = jnp.ones((16,), jnp.float32)

    @pl.loop(0, NP // NS, step=16)
    def _(i):
        zerov[pl.ds(i, 16)] = jnp.zeros((16,), jnp.float32)

    pltpu.sync_copy(zerov, acc.at[pl.ds(s * (NP // NS), NP // NS)])

    r0 = E // CH + DR * w
    nwaves = jnp.where(w < NW - 1, DR // 8, (2 * E // CH - (E // CH + DR * (NW - 1))) // 8)

    @pl.when(w < NW - 1)
    def _():
        pltpu.sync_copy(e2_hbm.at[pl.ds(r0, DR)], didx)

    @pl.when(w == NW - 1)
    def _():
        pltpu.sync_copy(e2_hbm.at[pl.ds(r0, 40)], didx.at[pl.ds(0, 40)])

    plsc.subcore_barrier()

    @pl.loop(0, DR // 8)
    def _(k):
        @pl.when(k < nwaves)
        def _():
            descs = [pltpu.async_copy(onesv, acc.at[didx.at[8 * k + t]],
                                      sem, add=True) for t in range(8)]
            for d in descs:
                d.wait()

    plsc.subcore_barrier()

    @pl.when(s == 0)
    def _():
        pltpu.sync_copy(acc, out_hbm.at[c])


@functools.partial(
    pl.kernel,
    out_type=jax.ShapeDtypeStruct((NC, NP, D), jnp.float32),
    mesh=_vmesh,
    scratch_types=[
        pltpu.VMEM_SHARED((NP, D), jnp.float32),
        [pltpu.VMEM((CH,), jnp.int32)] * NIB,
        [pltpu.VMEM((CH,), jnp.int32)] * NIB,
        pltpu.VMEM((CT,), jnp.int32),
        pltpu.VMEM((CT,), jnp.int32),
        [pltpu.VMEM((CH, D), jnp.float32)] * NBUF,
        [pltpu.SemaphoreType.DMA] * NIB,
        [pltpu.SemaphoreType.DMA] * NBUF,
        [pltpu.SemaphoreType.DMA] * NBUF,
    ],
)
def _agg_kernel(h_hbm, ef_hbm, out_hbm, acc, sbuf, dbuf, tsrc, tdst, rows,
                isem, gsem, ssem):
    c = lax.axis_index("c")
    s = lax.axis_index("s")
    w = s * NC + c

    @pl.loop(0, CH)
    def _(i):
        @pl.loop(0, D, step=16)
        def _(j):
            rows[0][i, pl.ds(j, 16)] = jnp.zeros((16,), jnp.float32)

    @pl.loop(0, NP // NS // CH)
    def _(k):
        pltpu.sync_copy(rows[0], acc.at[pl.ds(s * (NP // NS) + k * CH, CH)])

    plsc.subcore_barrier()

    def i_descs(j, bi):
        base = w * EW + j * CH
        return (pltpu.make_async_copy(ef_hbm.at[pl.ds(base, CH)],
                                      sbuf[bi], isem[bi]),
                pltpu.make_async_copy(ef_hbm.at[pl.ds(E + base, CH)],
                                      dbuf[bi], isem[bi]))

    def idx_start(j, bi):
        d1, d2 = i_descs(j, bi)
        d1.start()
        d2.start()

    def idx_wait(j, bi):
        d1, d2 = i_descs(j, bi)
        d1.wait()
        d2.wait()

    def g_desc(bi, b):
        return pltpu.make_async_copy(h_hbm.at[sbuf[bi]], rows[b], gsem[b])

    def s_desc(bi, b):
        return pltpu.make_async_copy(rows[b], acc.at[dbuf[bi]], ssem[b])

    idx_start(0, 0)
    idx_start(1, 1)
    idx_wait(0, 0)
    g_desc(0, 0).start()
    idx_start(2, 2)
    idx_wait(1, 1)
    g_desc(1, 1).start()

    @pl.loop(0, (CF - 12) // 12)
    def _(k):
        for b12 in range(12):
            j = 12 * k + b12
            bi, b = b12 % NIB, b12 % NBUF
            bi2, b2 = (b12 + 2) % NIB, (b12 + 2) % NBUF
            g_desc(bi, b).wait()
            s_desc(bi, b).start(add=True)
            idx_start(j + 3, (b12 + 3) % NIB)
            if b12 < 2:
                @pl.when(k > 0)
                def _():
                    s_desc((b12 + 4) % NIB, b2).wait()
            else:
                s_desc((b12 + 4) % NIB, b2).wait()
            idx_wait(j + 2, bi2)
            g_desc(bi2, b2).start()

    for j in range(CF - 12, CF):
        bi, b = j % NIB, j % NBUF
        bi2, b2 = (j + 2) % NIB, (j + 2) % NBUF
        g_desc(bi, b).wait()
        s_desc(bi, b).start(add=True)
        if j + 3 < CF:
            idx_start(j + 3, (j + 3) % NIB)
        s_desc((j - 2) % NIB, (j - 2) % NBUF).wait()
        if j + 2 < CF:
            idx_wait(j + 2, bi2)
            g_desc(bi2, b2).start()
    for j in (CF - 2, CF - 1):
        s_desc(j % NIB, j % NBUF).wait()

    tbase = w * EW + CF * CH
    pltpu.sync_copy(ef_hbm.at[pl.ds(tbase, CT)], tsrc)
    pltpu.sync_copy(ef_hbm.at[pl.ds(E + tbase, CT)], tdst)
    pltpu.sync_copy(h_hbm.at[tsrc], rows[0].at[pl.ds(0, CT)])
    pltpu.sync_copy(rows[0].at[pl.ds(0, CT)], acc.at[tdst], add=True)

    plsc.subcore_barrier()

    @pl.loop(0, NP // NS // CH)
    def _(k):
        r0 = s * (NP // NS) + k * CH
        pltpu.async_copy(acc.at[pl.ds(r0, CH)], out_hbm.at[c, pl.ds(r0, CH)],
                         isem[0])

    @pl.loop(0, NP // NS // CH)
    def _(k):
        r0 = s * (NP // NS) + k * CH
        pltpu.make_async_copy(acc.at[pl.ds(r0, CH)],
                              out_hbm.at[c, pl.ds(r0, CH)], isem[0]).wait()


BT2 = 2000
G5 = N // BT2


def _mm1(x, w1):
    def body(x_ref, w_ref, o_ref):
        o_ref[...] = jnp.dot(x_ref[...], w_ref[...],
                             preferred_element_type=jnp.float32)

    return pl.pallas_call(
        body,
        grid=(G5,),
        in_specs=[pl.BlockSpec((BT2, D), lambda i: (i, 0)),
                  pl.BlockSpec((D, D), lambda i: (0, 0))],
        out_specs=pl.BlockSpec((BT2, D), lambda i: (i, 0)),
        out_shape=jax.ShapeDtypeStruct((N, D), jnp.float32),
    )(x, w1)


def _prep(hraw, degt):
    def body(hraw_ref, degt_ref, h2_ref, dinv_ref):
        deg = degt_ref[:, 0:1] + degt_ref[:, 1:2] + 1.0
        dinv = lax.rsqrt(deg)
        h2_ref[...] = hraw_ref[...] * dinv
        dinv_ref[...] = dinv

    return pl.pallas_call(
        body,
        grid=(G5,),
        in_specs=[pl.BlockSpec((BT2, D), lambda i: (i, 0)),
                  pl.BlockSpec((BT2, NC), lambda i: (i, 0))],
        out_specs=[pl.BlockSpec((BT2, D), lambda i: (i, 0)),
                   pl.BlockSpec((BT2, 1), lambda i: (i, 0))],
        out_shape=[jax.ShapeDtypeStruct((N, D), jnp.float32),
                   jax.ShapeDtypeStruct((N, 1), jnp.float32)],
    )(hraw, degt)


def _layer1_finish(aggp, h2, dinv, b1, gamma, beta, w2):
    def body(aggp_ref, h2_ref, dinv_ref, b1_ref, g_ref, be_ref, w2_ref,
             h2b_ref, st_ref):
        i = pl.program_id(0)
        a = aggp_ref[0] + aggp_ref[1] + h2_ref[...]
        out1 = jnp.maximum(a * dinv_ref[...] + b1_ref[...], 0.0)

        @pl.when(i == 0)
        def _():
            st_ref[...] = jnp.zeros_like(st_ref)

        @pl.when(i < G5)
        def _():
            st_ref[0:1, :] += jnp.sum(out1, axis=0, keepdims=True)
            st_ref[1:2, :] += jnp.sum(out1 * out1, axis=0, keepdims=True)

        @pl.when(i >= G5)
        def _():
            mean = st_ref[0:1, :] * (1.0 / N)
            var = st_ref[1:2, :] * (1.0 / N) - mean * mean
            scale = g_ref[...] * lax.rsqrt(var + 1e-5)
            shift = be_ref[...] - mean * scale
            y = out1 * scale + shift
            h2b_ref[...] = jnp.dot(
                y, w2_ref[...],
                preferred_element_type=jnp.float32) * dinv_ref[...]

    blk = lambda i: (i % G5, 0)
    return pl.pallas_call(
        body,
        grid=(2 * G5,),
        in_specs=[pl.BlockSpec((NC, BT2, D), lambda i: (0, i % G5, 0)),
                  pl.BlockSpec((BT2, D), blk),
                  pl.BlockSpec((BT2, 1), blk),
                  pl.BlockSpec((1, D), lambda i: (0, 0)),
                  pl.BlockSpec((1, D), lambda i: (0, 0)),
                  pl.BlockSpec((1, D), lambda i: (0, 0)),
                  pl.BlockSpec((D, D), lambda i: (0, 0))],
        out_specs=[pl.BlockSpec((BT2, D), blk),
                   pl.BlockSpec((2, D), lambda i: (0, 0))],
        out_shape=[jax.ShapeDtypeStruct((N, D), jnp.float32),
                   jax.ShapeDtypeStruct((2, D), jnp.float32)],
    )(aggp, h2, dinv, b1, gamma, beta, w2)[0]


def _final(aggp, h2b, dinv, b2):
    def body(aggp_ref, h2b_ref, dinv_ref, b2_ref, o_ref):
        a = aggp_ref[0] + aggp_ref[1] + h2b_ref[...]
        o_ref[...] = jnp.maximum(a * dinv_ref[...] + b2_ref[...], 0.0)

    return pl.pallas_call(
        body,
        grid=(G5,),
        in_specs=[pl.BlockSpec((NC, BT2, D), lambda i: (0, i, 0)),
                  pl.BlockSpec((BT2, D), lambda i: (i, 0)),
                  pl.BlockSpec((BT2, 1), lambda i: (i, 0)),
                  pl.BlockSpec((1, D), lambda i: (0, 0))],
        out_specs=pl.BlockSpec((BT2, D), lambda i: (i, 0)),
        out_shape=jax.ShapeDtypeStruct((N, D), jnp.float32),
    )(aggp, h2b, dinv, b2)


def kernel(x, edge_index, W1, b1, gamma, beta, W2, b2):
    ef = edge_index.reshape(2 * E)
    e2 = edge_index.reshape(2 * E // CH, CH)
    b1r = b1.reshape(1, D)
    b2r = b2.reshape(1, D)
    gr = gamma.reshape(1, D)
    br = beta.reshape(1, D)

    degp = _deg_kernel(e2)
    hraw = _mm1(x, W1)
    degt = degp.T
    h2, dinv = _prep(hraw, degt[:N])
    aggp = _agg_kernel(h2, ef)
    h2b = _layer1_finish(aggp, h2, dinv, b1r, gr, br, W2)
    aggp2 = _agg_kernel(h2b, ef)
    return _final(aggp2, h2b, dinv, b2r)


# speedup vs baseline: 1.0093x; 1.0093x over previous
import functools

import jax
import jax.numpy as jnp
from jax import lax
from jax.experimental import pallas as pl
from jax.experimental.pallas import tpu as pltpu
from jax.experimental.pallas import tpu_sc as plsc

N = 10000
NP = 10240
E = 320000
D = 128
NC = 2
NS = 16
NW = NC * NS
CH = 64
EW = E // NW
CF = EW // CH
CT = EW - CF * CH
DR = 160
BT = 2048
NB = NP // BT
NBUF = 4
NIB = 6

_vmesh = plsc.VectorSubcoreMesh(core_axis_name="c", subcore_axis_name="s")


@functools.partial(
    pl.kernel,
    out_type=jax.ShapeDtypeStruct((NC, NP), jnp.float32),
    mesh=_vmesh,
    scratch_types=[
        pltpu.VMEM_SHARED((NP,), jnp.float32),
        pltpu.VMEM((DR, CH), jnp.int32),
        pltpu.VMEM((CH,), jnp.float32),
        pltpu.VMEM((NP // NS,), jnp.float32),
        pltpu.SemaphoreType.DMA,
    ],
)
def _deg_kernel(e2_hbm, out_hbm, acc, didx, onesv, zerov, sem):
    c = lax.axis_index("c")
    s = lax.axis_index("s")
    w = s * NC + c

    @pl.loop(0, CH, step=16)
    def _(i):
        onesv[pl.ds(i, 16)] = jnp.ones((16,), jnp.float32)

    @pl.loop(0, NP // NS, step=16)
    def _(i):
        zerov[pl.ds(i, 16)] = jnp.zeros((16,), jnp.float32)

    pltpu.sync_copy(zerov, acc.at[pl.ds(s * (NP // NS), NP // NS)])

    r0 = E // CH + DR * w
    nwaves = jnp.where(w < NW - 1, DR // 8, (2 * E // CH - (E // CH + DR * (NW - 1))) // 8)

    @pl.when(w < NW - 1)
    def _():
        pltpu.sync_copy(e2_hbm.at[pl.ds(r0, DR)], didx)

    @pl.when(w == NW - 1)
    def _():
        pltpu.sync_copy(e2_hbm.at[pl.ds(r0, 40)], didx.at[pl.ds(0, 40)])

    plsc.subcore_barrier()

    @pl.loop(0, DR // 8)
    def _(k):
        @pl.when(k < nwaves)
        def _():
            descs = [pltpu.async_copy(onesv, acc.at[didx.at[8 * k + t]],
                                      sem, add=True) for t in range(8)]
            for d in descs:
                d.wait()

    plsc.subcore_barrier()

    @pl.when(s == 0)
    def _():
        pltpu.sync_copy(acc, out_hbm.at[c])


@functools.partial(
    pl.kernel,
    out_type=jax.ShapeDtypeStruct((NC, NP, D), jnp.float32),
    mesh=_vmesh,
    scratch_types=[
        pltpu.VMEM_SHARED((NP, D), jnp.float32),
        [pltpu.VMEM((CH,), jnp.int32)] * NIB,
        [pltpu.VMEM((CH,), jnp.int32)] * NIB,
        pltpu.VMEM((CT,), jnp.int32),
        pltpu.VMEM((CT,), jnp.int32),
        [pltpu.VMEM((CH, D), jnp.float32)] * NBUF,
        [pltpu.SemaphoreType.DMA] * NIB,
        [pltpu.SemaphoreType.DMA] * NBUF,
        [pltpu.SemaphoreType.DMA] * NBUF,
    ],
)
def _agg_kernel(h_hbm, ef_hbm, out_hbm, acc, sbuf, dbuf, tsrc, tdst, rows,
                isem, gsem, ssem):
    c = lax.axis_index("c")
    s = lax.axis_index("s")
    w = s * NC + c

    @pl.loop(0, CH)
    def _(i):
        @pl.loop(0, D, step=16)
        def _(j):
            rows[0][i, pl.ds(j, 16)] = jnp.zeros((16,), jnp.float32)

    @pl.loop(0, NP // NS // CH)
    def _(k):
        pltpu.sync_copy(rows[0], acc.at[pl.ds(s * (NP // NS) + k * CH, CH)])

    plsc.subcore_barrier()

    def i_descs(j, bi):
        base = w * EW + j * CH
        return (pltpu.make_async_copy(ef_hbm.at[pl.ds(base, CH)],
                                      sbuf[bi], isem[bi]),
                pltpu.make_async_copy(ef_hbm.at[pl.ds(E + base, CH)],
                                      dbuf[bi], isem[bi]))

    def idx_start(j, bi):
        d1, d2 = i_descs(j, bi)
        d1.start()
        d2.start()

    def idx_wait(j, bi):
        d1, d2 = i_descs(j, bi)
        d1.wait()
        d2.wait()

    def g_desc(bi, b):
        return pltpu.make_async_copy(h_hbm.at[sbuf[bi]], rows[b], gsem[b])

    def s_desc(bi, b):
        return pltpu.make_async_copy(rows[b], acc.at[dbuf[bi]], ssem[b])

    idx_start(0, 0)
    idx_start(1, 1)
    idx_wait(0, 0)
    g_desc(0, 0).start()
    idx_start(2, 2)
    idx_wait(1, 1)
    g_desc(1, 1).start()

    @pl.loop(0, (CF - 12) // 12)
    def _(k):
        for b12 in range(12):
            j = 12 * k + b12
            bi, b = b12 % NIB, b12 % NBUF
            bi2, b2 = (b12 + 2) % NIB, (b12 + 2) % NBUF
            g_desc(bi, b).wait()
            s_desc(bi, b).start(add=True)
            idx_start(j + 3, (b12 + 3) % NIB)
            if b12 < 2:
                @pl.when(k > 0)
                def _():
                    s_desc((b12 + 4) % NIB, b2).wait()
            else:
                s_desc((b12 + 4) % NIB, b2).wait()
            idx_wait(j + 2, bi2)
            g_desc(bi2, b2).start()

    for j in range(CF - 12, CF):
        bi, b = j % NIB, j % NBUF
        bi2, b2 = (j + 2) % NIB, (j + 2) % NBUF
        g_desc(bi, b).wait()
        s_desc(bi, b).start(add=True)
        if j + 3 < CF:
            idx_start(j + 3, (j + 3) % NIB)
        s_desc((j - 2) % NIB, (j - 2) % NBUF).wait()
        if j + 2 < CF:
            idx_wait(j + 2, bi2)
            g_desc(bi2, b2).start()
    for j in (CF - 2, CF - 1):
        s_desc(j % NIB, j % NBUF).wait()

    tbase = w * EW + CF * CH
    pltpu.sync_copy(ef_hbm.at[pl.ds(tbase, CT)], tsrc)
    pltpu.sync_copy(ef_hbm.at[pl.ds(E + tbase, CT)], tdst)
    pltpu.sync_copy(h_hbm.at[tsrc], rows[0].at[pl.ds(0, CT)])
    pltpu.sync_copy(rows[0].at[pl.ds(0, CT)], acc.at[tdst], add=True)

    plsc.subcore_barrier()

    @pl.loop(0, NP // NS // CH)
    def _(k):
        r0 = s * (NP // NS) + k * CH
        pltpu.async_copy(acc.at[pl.ds(r0, CH)], out_hbm.at[c, pl.ds(r0, CH)],
                         isem[0])

    @pl.loop(0, NP // NS // CH)
    def _(k):
        r0 = s * (NP // NS) + k * CH
        pltpu.make_async_copy(acc.at[pl.ds(r0, CH)],
                              out_hbm.at[c, pl.ds(r0, CH)], isem[0]).wait()


BT2 = 2000
G5 = N // BT2


def _mm_prep(x, w1, degt):
    def body(x_ref, w_ref, degt_ref, h2_ref, dinv_ref):
        deg = degt_ref[:, 0:1] + degt_ref[:, 1:2] + 1.0
        dinv = lax.rsqrt(deg)
        h2_ref[...] = jnp.dot(x_ref[...], w_ref[...],
                              preferred_element_type=jnp.float32) * dinv
        dinv_ref[...] = dinv

    return pl.pallas_call(
        body,
        grid=(G5,),
        in_specs=[pl.BlockSpec((BT2, D), lambda i: (i, 0)),
                  pl.BlockSpec((D, D), lambda i: (0, 0)),
                  pl.BlockSpec((BT2, NC), lambda i: (i, 0))],
        out_specs=[pl.BlockSpec((BT2, D), lambda i: (i, 0)),
                   pl.BlockSpec((BT2, 1), lambda i: (i, 0))],
        out_shape=[jax.ShapeDtypeStruct((N, D), jnp.float32),
                   jax.ShapeDtypeStruct((N, 1), jnp.float32)],
    )(x, w1, degt)


def _layer1_finish(aggp, h2, dinv, b1, gamma, beta, w2):
    def body(aggp_ref, h2_ref, dinv_ref, b1_ref, g_ref, be_ref, w2_ref,
             h2b_ref, st_ref):
        i = pl.program_id(0)
        a = aggp_ref[0] + aggp_ref[1] + h2_ref[...]
        out1 = jnp.maximum(a * dinv_ref[...] + b1_ref[...], 0.0)

        @pl.when(i == 0)
        def _():
            st_ref[...] = jnp.zeros_like(st_ref)

        @pl.when(i < G5)
        def _():
            st_ref[0:1, :] += jnp.sum(out1, axis=0, keepdims=True)
            st_ref[1:2, :] += jnp.sum(out1 * out1, axis=0, keepdims=True)

        @pl.when(i >= G5)
        def _():
            mean = st_ref[0:1, :] * (1.0 / N)
            var = st_ref[1:2, :] * (1.0 / N) - mean * mean
            scale = g_ref[...] * lax.rsqrt(var + 1e-5)
            shift = be_ref[...] - mean * scale
            y = out1 * scale + shift
            h2b_ref[...] = jnp.dot(
                y, w2_ref[...],
                preferred_element_type=jnp.float32) * dinv_ref[...]

    blk = lambda i: (i % G5, 0)
    return pl.pallas_call(
        body,
        grid=(2 * G5,),
        in_specs=[pl.BlockSpec((NC, BT2, D), lambda i: (0, i % G5, 0)),
                  pl.BlockSpec((BT2, D), blk),
                  pl.BlockSpec((BT2, 1), blk),
                  pl.BlockSpec((1, D), lambda i: (0, 0)),
                  pl.BlockSpec((1, D), lambda i: (0, 0)),
                  pl.BlockSpec((1, D), lambda i: (0, 0)),
                  pl.BlockSpec((D, D), lambda i: (0, 0))],
        out_specs=[pl.BlockSpec((BT2, D), blk),
                   pl.BlockSpec((2, D), lambda i: (0, 0))],
        out_shape=[jax.ShapeDtypeStruct((N, D), jnp.float32),
                   jax.ShapeDtypeStruct((2, D), jnp.float32)],
    )(aggp, h2, dinv, b1, gamma, beta, w2)[0]


def _final(aggp, h2b, dinv, b2):
    def body(aggp_ref, h2b_ref, dinv_ref, b2_ref, o_ref):
        a = aggp_ref[0] + aggp_ref[1] + h2b_ref[...]
        o_ref[...] = jnp.maximum(a * dinv_ref[...] + b2_ref[...], 0.0)

    return pl.pallas_call(
        body,
        grid=(G5,),
        in_specs=[pl.BlockSpec((NC, BT2, D), lambda i: (0, i, 0)),
                  pl.BlockSpec((BT2, D), lambda i: (i, 0)),
                  pl.BlockSpec((BT2, 1), lambda i: (i, 0)),
                  pl.BlockSpec((1, D), lambda i: (0, 0))],
        out_specs=pl.BlockSpec((BT2, D), lambda i: (i, 0)),
        out_shape=jax.ShapeDtypeStruct((N, D), jnp.float32),
    )(aggp, h2b, dinv, b2)


def kernel(x, edge_index, W1, b1, gamma, beta, W2, b2):
    ef = edge_index.reshape(2 * E)
    e2 = edge_index.reshape(2 * E // CH, CH)
    b1r = b1.reshape(1, D)
    b2r = b2.reshape(1, D)
    gr = gamma.reshape(1, D)
    br = beta.reshape(1, D)

    degp = _deg_kernel(e2)
    degt = degp.T
    h2, dinv = _mm_prep(x, W1, degt[:N])
    aggp = _agg_kernel(h2, ef)
    h2b = _layer1_finish(aggp, h2, dinv, b1r, gr, br, W2)
    aggp2 = _agg_kernel(h2b, ef)
    return _final(aggp2, h2b, dinv, b2r)
